# Initial kernel scaffold; baseline (speedup 1.0000x reference)
import jax
import jax.numpy as jnp
from jax.experimental import pallas as pl


def kernel(edge_index, user_indices, item_indices, user_emb, item_emb, W1, b1, W2, b2, W3, b3, P1, pb1, P2, pb2, P3, pb3):
    raise NotImplementedError("write your pallas kernel here")


# speedup vs baseline: 4.9315x; 4.9315x over previous
import functools

import jax
import jax.numpy as jnp
from jax import lax
from jax.experimental import pallas as pl
from jax.experimental.pallas import tpu as pltpu
from jax.experimental.pallas import tpu_sc as plsc

N_USERS = 10000
N_ITEMS = 40000
N_NODES = 50000
N_EDGES = 800000
D = 64
BATCH = 4096

NC = 2
NS = 16
L = 16

HALF = N_NODES // NC
ACC_ROWS = 26624
SLAB = ACC_ROWS // NS
DUMMY = 26000

CHUNK = 256
CROWS = CHUNK // 128
NCHUNK = N_EDGES // CHUNK
DC, DREM = NCHUNK // 32, NCHUNK % 32
AC, AREM = NCHUNK // NS, NCHUNK % NS

DEGP = 50048
DSL = DEGP // NS

_mesh = functools.partial(
    plsc.VectorSubcoreMesh,
    core_axis_name="c", subcore_axis_name="s",
    num_cores=NC, num_subcores=NS,
)


def _fill(ref, row, n, val):
    v = jnp.full((L,), val, ref.dtype)
    for j in range(n // L):
        if row is None:
            ref[pl.ds(j * L, L)] = v
        else:
            ref[row, pl.ds(j * L, L)] = v


def _deg_body(src_h, dst_h, out_h, dout_s, din_s, zbuf, ones, sidx, didx):
    c = lax.axis_index("c")
    s = lax.axis_index("s")
    w = s * NC + c

    _fill(zbuf, None, DSL, 0.0)
    _fill(ones, None, 128, 1.0)
    pltpu.sync_copy(zbuf.at[pl.ds(0, DSL)], dout_s.at[pl.ds(s * DSL, DSL)])
    pltpu.sync_copy(zbuf.at[pl.ds(0, DSL)], din_s.at[pl.ds(s * DSL, DSL)])
    plsc.subcore_barrier()

    count = DC + jnp.where(w < DREM, 1, 0)

    def chunk_step(i, _):
        cid = w + 32 * i
        pltpu.sync_copy(src_h.at[cid], sidx)
        pltpu.sync_copy(dst_h.at[cid], didx)
        for k in range(CROWS):
            pltpu.sync_copy(ones, dout_s.at[sidx.at[k]], add=True)
            pltpu.sync_copy(ones, din_s.at[didx.at[k]], add=True)
        return _

    lax.fori_loop(0, count, chunk_step, None)
    plsc.subcore_barrier()

    pltpu.sync_copy(dout_s.at[pl.ds(s * DSL, DSL)], zbuf)
    pltpu.sync_copy(zbuf, out_h.at[pl.ds((c * 2) * DEGP + s * DSL, DSL)])
    pltpu.sync_copy(din_s.at[pl.ds(s * DSL, DSL)], zbuf)
    pltpu.sync_copy(zbuf, out_h.at[pl.ds((c * 2 + 1) * DEGP + s * DSL, DSL)])


def _deg_sc(src3d, dst3d):
    return pl.kernel(
        _deg_body,
        out_type=jax.ShapeDtypeStruct((NC * 2 * DEGP,), jnp.float32),
        mesh=_mesh(),
        scratch_types=[
            pltpu.VMEM_SHARED((DEGP,), jnp.float32),
            pltpu.VMEM_SHARED((DEGP,), jnp.float32),
            pltpu.VMEM((DSL,), jnp.float32),
            pltpu.VMEM((128,), jnp.float32),
            pltpu.VMEM((CROWS, 128), jnp.int32),
            pltpu.VMEM((CROWS, 128), jnp.int32),
        ],
    )(src3d, dst3d)


def _agg_body(h_h, src3d, dst3d, out_h, acc, sidx, didx, ldst, rows, sem):
    c = lax.axis_index("c")
    s = lax.axis_index("s")
    lo = c * HALF

    def _zrow(r, _):
        for j in range(D // L):
            rows[r, pl.ds(j * L, L)] = jnp.zeros((L,), jnp.float32)
        return _
    lax.fori_loop(0, 128, _zrow, None)
    zb = rows.at[pl.ds(0, 128)]
    for j in range(SLAB // 128):
        pltpu.sync_copy(zb, acc.at[pl.ds(s * SLAB + j * 128, 128)])
    plsc.subcore_barrier()

    lane = lax.iota(jnp.int32, L)
    count = AC + jnp.where(s < AREM, 1, 0)

    def chunk_step(i, _):
        cid = s + NS * i
        pltpu.sync_copy(src3d.at[cid], sidx)
        pltpu.sync_copy(dst3d.at[cid], didx)
        for k in range(CROWS):
            for j in range(128 // L):
                d = didx[k, pl.ds(j * L, L)]
                ok = (d >= lo) & (d < lo + HALF)
                ldst[k, pl.ds(j * L, L)] = jnp.where(
                    ok, d - lo, DUMMY + k * L + lane)
        cps = [pltpu.async_copy(h_h.at[sidx.at[k]],
                                rows.at[pl.ds(k * 128, 128)], sem)
               for k in range(CROWS)]
        for cp in cps:
            cp.wait()
        for k in range(CROWS):
            pltpu.sync_copy(rows.at[pl.ds(k * 128, 128)],
                            acc.at[ldst.at[k]], add=True)
        return _

    lax.fori_loop(0, count, chunk_step, None)
    plsc.subcore_barrier()

    bnc = rows.at[pl.ds(0, 128)]
    for j in range(SLAB // 128):
        r = s * SLAB + j * 128
        @pl.when(r + 128 <= HALF)
        def _w():
            pltpu.sync_copy(acc.at[pl.ds(r, 128)], bnc)
            pltpu.sync_copy(bnc, out_h.at[pl.ds(lo + r, 128)])
    @pl.when(s == NS - 1)
    def _wt():
        r = HALF - 40
        pltpu.sync_copy(acc.at[pl.ds(r, 40)], rows.at[pl.ds(0, 40)])
        pltpu.sync_copy(rows.at[pl.ds(0, 40)], out_h.at[pl.ds(lo + r, 40)])


def _agg_sc(h, src3d, dst3d):
    return pl.kernel(
        _agg_body,
        out_type=jax.ShapeDtypeStruct((N_NODES, D), jnp.float32),
        mesh=_mesh(),
        scratch_types=[
            pltpu.VMEM_SHARED((ACC_ROWS, D), jnp.float32),
            pltpu.VMEM((CROWS, 128), jnp.int32),
            pltpu.VMEM((CROWS, 128), jnp.int32),
            pltpu.VMEM((CROWS, 128), jnp.int32),
            pltpu.VMEM((CHUNK, D), jnp.float32),
            pltpu.SemaphoreType.DMA,
        ],
        compiler_params=pltpu.CompilerParams(use_tc_tiling_on_sc=False),
    )(h, src3d, dst3d)


def _gather_body(x_h, deg_h, idx_h, rows_o, deg_o, iidx, rbuf, dbuf, sem):
    c = lax.axis_index("c")
    s = lax.axis_index("s")
    w = s * NC + c
    pltpu.sync_copy(idx_h.at[pl.ds(w * 2, 2)], iidx)
    for k in range(2):
        base = w * 256 + k * 128
        pltpu.async_copy(x_h.at[iidx.at[k]], rbuf, sem).wait()
        pltpu.async_copy(deg_h.at[iidx.at[k]], dbuf, sem).wait()
        pltpu.sync_copy(rbuf, rows_o.at[pl.ds(base, 128)])
        pltpu.sync_copy(dbuf, deg_o.at[pl.ds(base, 128)])


def _gather_sc(x, deg_flat, idx2d):
    nrow = idx2d.shape[0] * 128
    return pl.kernel(
        _gather_body,
        out_type=(jax.ShapeDtypeStruct((nrow, D), jnp.float32),
                  jax.ShapeDtypeStruct((nrow,), jnp.float32)),
        mesh=_mesh(),
        scratch_types=[
            pltpu.VMEM((2, 128), jnp.int32),
            pltpu.VMEM((128, D), jnp.float32),
            pltpu.VMEM((128,), jnp.float32),
            pltpu.SemaphoreType.DMA,
        ],
        compiler_params=pltpu.CompilerParams(use_tc_tiling_on_sc=False),
    )(x, deg_flat, idx2d)


BLK = 2000


def _norm(deg):
    return jnp.where(deg > 0, lax.rsqrt(jnp.maximum(deg, 1.0)), 0.0)


def _layer1_tc_body(x_r, do_r, w_r, o_r):
    ns = _norm(do_r[...])
    o_r[...] = jnp.dot(x_r[...] * ns, w_r[...],
                       preferred_element_type=jnp.float32)


def _layer_tc_body(a_r, do_r, di_r, b_r, w_r, o_r):
    ns = _norm(do_r[...])
    nd = _norm(di_r[...])
    t = jnp.maximum(a_r[...] * nd + b_r[...], 0.0)
    o_r[...] = jnp.dot(t * ns, w_r[...], preferred_element_type=jnp.float32)


def _layer_tc(a, do, di, b, w, first=False):
    grid = (N_NODES // BLK,)
    row = pl.BlockSpec((BLK, D), lambda i: (i, 0))
    deg = pl.BlockSpec((BLK, 1), lambda i: (i, 0))
    wsp = pl.BlockSpec((D, D), lambda i: (0, 0))
    bsp = pl.BlockSpec((1, D), lambda i: (0, 0))
    if first:
        return pl.pallas_call(
            _layer1_tc_body, grid=grid,
            in_specs=[row, deg, wsp],
            out_specs=row,
            out_shape=jax.ShapeDtypeStruct((N_NODES, D), jnp.float32),
        )(a, do, w)
    return pl.pallas_call(
        _layer_tc_body, grid=grid,
        in_specs=[row, deg, deg, bsp, wsp],
        out_specs=row,
        out_shape=jax.ShapeDtypeStruct((N_NODES, D), jnp.float32),
    )(a, do, di, b, w)


MBLK = 1024


def _mlp_body(u_r, i_r, du_r, di_r, b3_r, p1u_r, p1i_r, pb1_r, p2_r, pb2_r,
              p3_r, pb3_r, o_r):
    xu = u_r[...] * _norm(du_r[...]) + b3_r[...]
    xi = i_r[...] * _norm(di_r[...]) + b3_r[...]
    h = jnp.dot(xu, p1u_r[...], preferred_element_type=jnp.float32)
    h = h + jnp.dot(xi, p1i_r[...], preferred_element_type=jnp.float32)
    h = jnp.maximum(h + pb1_r[...], 0.0)
    h = jnp.maximum(jnp.dot(h, p2_r[...], preferred_element_type=jnp.float32)
                    + pb2_r[...], 0.0)
    z = jnp.dot(h, p3_r[...], preferred_element_type=jnp.float32) + pb3_r[...]
    o_r[...] = 1.0 / (1.0 + jnp.exp(-z))


def _mlp_tc(rows, dg, b3, P1, pb1, P2, pb2, P3, pb3):
    grid = (BATCH // MBLK,)
    u = pl.BlockSpec((MBLK, D), lambda i: (i, 0))
    it = pl.BlockSpec((MBLK, D), lambda i: (i + BATCH // MBLK, 0))
    du = pl.BlockSpec((MBLK, 1), lambda i: (i, 0))
    di = pl.BlockSpec((MBLK, 1), lambda i: (i + BATCH // MBLK, 0))
    full = lambda a, b: pl.BlockSpec((a, b), lambda i: (0, 0))
    return pl.pallas_call(
        _mlp_body, grid=grid,
        in_specs=[u, it, du, di, full(1, D), full(D, D), full(D, D),
                  full(1, D), full(D, 32), full(1, 32), full(32, 1),
                  full(1, 1)],
        out_specs=pl.BlockSpec((MBLK, 1), lambda i: (i, 0)),
        out_shape=jax.ShapeDtypeStruct((BATCH, 1), jnp.float32),
    )(rows, rows, dg, dg, b3, P1[:D], P1[D:], pb1, P2, pb2, P3, pb3)


def kernel(edge_index, user_indices, item_indices, user_emb, item_emb,
           W1, b1, W2, b2, W3, b3, P1, pb1, P2, pb2, P3, pb3):
    src3d = edge_index[0].reshape(NCHUNK, CROWS, 128).astype(jnp.int32)
    dst3d = edge_index[1].reshape(NCHUNK, CROWS, 128).astype(jnp.int32)
    x0 = jnp.concatenate([user_emb, item_emb], axis=0)

    degp = _deg_sc(src3d, dst3d).reshape(NC, 2, DEGP)
    deg_out = (degp[0, 0, :N_NODES] + degp[1, 0, :N_NODES]).reshape(N_NODES, 1)
    deg_in_f = degp[0, 1, :N_NODES] + degp[1, 1, :N_NODES]
    deg_in = deg_in_f.reshape(N_NODES, 1)

    b1r = b1.reshape(1, D)
    b2r = b2.reshape(1, D)
    b3r = b3.reshape(1, D)

    h = _layer_tc(x0, deg_out, None, None, W1, first=True)
    a = _agg_sc(h, src3d, dst3d)
    h = _layer_tc(a, deg_out, deg_in, b1r, W2)
    a = _agg_sc(h, src3d, dst3d)
    h = _layer_tc(a, deg_out, deg_in, b2r, W3)
    a = _agg_sc(h, src3d, dst3d)

    idx = jnp.concatenate([user_indices.astype(jnp.int32),
                           (N_USERS + item_indices).astype(jnp.int32)])
    idx2d = idx.reshape(2 * BATCH // 128, 128)
    rows, dg = _gather_sc(a, deg_in_f, idx2d)

    pred = _mlp_tc(rows, dg.reshape(2 * BATCH, 1), b3r,
                   P1, pb1.reshape(1, 32 * 2), P2, pb2.reshape(1, 32),
                   P3, pb3.reshape(1, 1))
    return pred
